# Initial kernel scaffold; baseline (speedup 1.0000x reference)
import jax
import jax.numpy as jnp
from jax.experimental import pallas as pl


def kernel(x, edge_index, node_id, Wl1, bl1, Wr1, br1, att1, b1, Wl2, bl2, Wr2, br2, att2, b2, Wl3, bl3, Wr3, br3, att3, b3, D1w, D1b, D2w, D2b):
    raise NotImplementedError("write your pallas kernel here")


# speedup vs baseline: 26.0766x; 26.0766x over previous
import functools

import jax
import jax.numpy as jnp
from jax import lax
from jax.experimental import pallas as pl
from jax.experimental.pallas import tpu as pltpu
from jax.experimental.pallas import tpu_sc as plsc

N = 10000
E = 320000
H = 20
HP = 32
NACC = N + 112
NC, NS = 2, 16
NW = NC * NS
ZR = NACC // NS

EP = E + N
KCH = 81
PW = KCH * 128
EPAD = NW * PW

EDV = E // NW
KD = 79
PWD = KD * 128

_HIGH = jax.lax.Precision.HIGHEST


@functools.lru_cache(maxsize=1)
def _sc_mesh():
    return plsc.VectorSubcoreMesh(core_axis_name="c", subcore_axis_name="s",
                                  num_cores=NC, num_subcores=NS)


def _dot(a, b):
    return jnp.dot(a.astype(jnp.bfloat16), b.astype(jnp.bfloat16),
                   preferred_element_type=jnp.float32)


def _proj_first(x, wlp, blp, wrp, brp):
    def body(x_ref, wl_ref, bl_ref, wr_ref, br_ref, xl_ref, xr_ref, cm_ref):
        xv = x_ref[...]
        xl = _dot(xv, wl_ref[...]) + bl_ref[...]
        xr = _dot(xv, wr_ref[...]) + br_ref[...]
        xl_ref[...] = xl
        xr_ref[...] = xr
        ml = jnp.max(jnp.abs(xl), axis=0, keepdims=True)
        mr = jnp.max(jnp.abs(xr), axis=0, keepdims=True)
        cm_ref[...] = jnp.concatenate(
            [ml, mr, jnp.zeros((6, HP), jnp.float32)], axis=0)

    return pl.pallas_call(
        body,
        out_shape=[
            jax.ShapeDtypeStruct((N, HP), jnp.float32),
            jax.ShapeDtypeStruct((N, HP), jnp.float32),
            jax.ShapeDtypeStruct((8, HP), jnp.float32),
        ],
    )(x, wlp, blp, wrp, brp)


def _combine_proj(acc, bp, wlp, blp, wrp, brp):
    def body(acc_ref, b_ref, wl_ref, bl_ref, wr_ref, br_ref,
             x_ref, xl_ref, xr_ref, cm_ref):
        a = acc_ref[0:N, :] + acc_ref[NACC:NACC + N, :]
        s = a[:, 20:21]
        xn = jnp.maximum(a / s + b_ref[...], 0.0)
        x_ref[...] = xn
        xl = _dot(xn, wl_ref[...]) + bl_ref[...]
        xr = _dot(xn, wr_ref[...]) + br_ref[...]
        xl_ref[...] = xl
        xr_ref[...] = xr
        ml = jnp.max(jnp.abs(xl), axis=0, keepdims=True)
        mr = jnp.max(jnp.abs(xr), axis=0, keepdims=True)
        cm_ref[...] = jnp.concatenate(
            [ml, mr, jnp.zeros((6, HP), jnp.float32)], axis=0)

    return pl.pallas_call(
        body,
        out_shape=[
            jax.ShapeDtypeStruct((N, HP), jnp.float32),
            jax.ShapeDtypeStruct((N, HP), jnp.float32),
            jax.ShapeDtypeStruct((N, HP), jnp.float32),
            jax.ShapeDtypeStruct((8, HP), jnp.float32),
        ],
    )(acc, bp, wlp, blp, wrp, brp)


def _final_combine(acc, bp, x1p, x2p, a1, a2, a3, b1, b2, b3):
    BLK = 2000

    def body(accA_ref, accB_ref, b_ref, x1_ref, x2_ref,
             a1_ref, a2_ref, a3_ref, b1_ref, b2_ref, b3_ref,
             x3_ref, p_ref, q_ref):
        a = accA_ref[0] + accB_ref[0]
        s = a[:, 20:21]
        x3 = jnp.maximum(a / s + b_ref[...], 0.0)
        x3_ref[...] = x3
        x1 = x1_ref[...]
        x2 = x2_ref[...]
        p_ref[...] = _dot(x1, a1_ref[...]) + _dot(x2, a2_ref[...]) + _dot(x3, a3_ref[...])
        q_ref[...] = _dot(x1, b1_ref[...]) + _dot(x2, b2_ref[...]) + _dot(x3, b3_ref[...])

    acc2 = acc.reshape(2, NACC, HP)
    full = lambda shp: pl.BlockSpec(shp, lambda i: tuple(0 for _ in shp))
    return pl.pallas_call(
        body,
        grid=(N // BLK,),
        in_specs=[
            pl.BlockSpec((1, BLK, HP), lambda i: (0, i, 0)),
            pl.BlockSpec((1, BLK, HP), lambda i: (1, i, 0)),
            full((1, HP)),
            pl.BlockSpec((BLK, HP), lambda i: (i, 0)),
            pl.BlockSpec((BLK, HP), lambda i: (i, 0)),
            full((HP, 64)), full((HP, 64)), full((HP, 64)),
            full((HP, 64)), full((HP, 64)), full((HP, 64)),
        ],
        out_specs=[
            pl.BlockSpec((BLK, HP), lambda i: (i, 0)),
            pl.BlockSpec((BLK, 64), lambda i: (i, 0)),
            pl.BlockSpec((BLK, 64), lambda i: (i, 0)),
        ],
        out_shape=[
            jax.ShapeDtypeStruct((N, HP), jnp.float32),
            jax.ShapeDtypeStruct((N, 64), jnp.float32),
            jax.ShapeDtypeStruct((N, 64), jnp.float32),
        ],
    )(acc2, acc2, bp, x1p, x2p, a1, a2, a3, b1, b2, b3)


def _sparsemax(z):
    def body(z_ref, o_ref):
        zv = z_ref[...]
        zmax = jnp.max(zv)

        def bis(_, lh):
            lo, hi = lh
            mid = 0.5 * (lo + hi)
            f = jnp.sum(jnp.maximum(zv - mid, 0.0)) - 1.0
            good = f > 0.0
            return jnp.where(good, mid, lo), jnp.where(good, hi, mid)

        lo, hi = lax.fori_loop(0, 40, bis, (zmax - 1.0, zmax))
        ta = 0.5 * (lo + hi)
        sup = zv > ta
        k = jnp.sum(sup.astype(jnp.float32))
        tau = (jnp.sum(jnp.where(sup, zv, 0.0)) - 1.0) / k
        o_ref[...] = jnp.maximum(zv - tau, 0.0)

    return pl.pallas_call(
        body,
        out_shape=jax.ShapeDtypeStruct((NW, PWD), jnp.float32),
    )(z)


def _bfr(v):
    u = plsc.bitcast(v, jnp.uint32)
    r = (u + jnp.uint32(0x7FFF) + ((u >> jnp.uint32(16)) & jnp.uint32(1))) \
        & jnp.uint32(0xFFFF0000)
    return plsc.bitcast(r, jnp.float32)


@functools.lru_cache(maxsize=1)
def _build_gnn_edge():
  return functools.partial(
    pl.kernel,
    out_type=jax.ShapeDtypeStruct((2 * NACC, HP), jnp.float32),
    mesh=_sc_mesh(),
    compiler_params=pltpu.CompilerParams(needs_layout_passes=False, use_tc_tiling_on_sc=False),
    scratch_types=[
        pltpu.VMEM((KCH, 128), jnp.int32),
        pltpu.VMEM((KCH, 128), jnp.int32),
        pltpu.VMEM((128, HP), jnp.float32),
        pltpu.VMEM((128, HP), jnp.float32),
        pltpu.VMEM((128, HP), jnp.float32),
        pltpu.VMEM((HP,), jnp.float32),
        pltpu.VMEM((16,), jnp.float32),
        pltpu.VMEM((ZR, HP), jnp.float32),
        pltpu.VMEM_SHARED((NACC, HP), jnp.float32),
        pltpu.VMEM_SHARED((N, HP), jnp.float32),
        pltpu.VMEM_SHARED((N, HP), jnp.float32),
        pltpu.SemaphoreType.DMA,
        pltpu.SemaphoreType.DMA,
    ],
  )(_gnn_edge_body)


def _gnn_edge(*args):
    return _build_gnn_edge()(*args)


def _gnn_edge_body(xl_h, xr_h, att_h, mv_h, src_h, dst_h, out_h,
              src_t, dst_t, xlg, xrg, vbuf, attv, mvr, stage, acc, xls, xrs,
              sem1, sem2):
    cid = lax.axis_index("c")
    sid = lax.axis_index("s")
    wid = cid * NS + sid

    zer = jnp.zeros((16,), jnp.float32)

    def zbody(i, _):
        stage[i, pl.ds(0, 16)] = zer
        stage[i, pl.ds(16, 16)] = zer
        return 0

    lax.fori_loop(0, ZR, zbody, 0)
    pltpu.sync_copy(stage, acc.at[pl.ds(sid * ZR, ZR)])

    @pl.when(sid < NS - 1)
    def _():
        pltpu.sync_copy(xl_h.at[pl.ds(sid * ZR, ZR)], xls.at[pl.ds(sid * ZR, ZR)])
        pltpu.sync_copy(xr_h.at[pl.ds(sid * ZR, ZR)], xrs.at[pl.ds(sid * ZR, ZR)])

    @pl.when(sid == NS - 1)
    def _():
        tail = N - (NS - 1) * ZR
        pltpu.sync_copy(xl_h.at[pl.ds((NS - 1) * ZR, tail)],
                        xls.at[pl.ds((NS - 1) * ZR, tail)])
        pltpu.sync_copy(xr_h.at[pl.ds((NS - 1) * ZR, tail)],
                        xrs.at[pl.ds((NS - 1) * ZR, tail)])

    pltpu.sync_copy(src_h.at[wid], src_t)
    pltpu.sync_copy(dst_h.at[wid], dst_t)
    pltpu.sync_copy(att_h, attv)
    pltpu.sync_copy(mv_h, mvr)
    plsc.subcore_barrier()

    att_lo = attv[pl.ds(0, 16)]
    att_hi = attv[pl.ds(16, 16)]
    mvec = mvr[...]
    lane = lax.iota(jnp.int32, 16)

    def chunk(j, _):
        g1 = pltpu.async_copy(xls.at[src_t.at[j]], xlg, sem1)
        g2 = pltpu.async_copy(xrs.at[dst_t.at[j]], xrg, sem2)
        g1.wait()
        g2.wait()

        def grp(g, _):
            base = g * 16
            ev = jnp.zeros((16,), jnp.float32)
            for i in range(16):
                ii = base + i
                t0 = xlg[ii, pl.ds(0, 16)] + xrg[ii, pl.ds(0, 16)]
                t1 = xlg[ii, pl.ds(16, 16)] + xrg[ii, pl.ds(16, 16)]
                l0 = jnp.maximum(t0, 0.0) + 0.2 * jnp.minimum(t0, 0.0)
                l1 = jnp.maximum(t1, 0.0) + 0.2 * jnp.minimum(t1, 0.0)
                ev = jnp.where(
                    lane == i,
                    jnp.sum(_bfr(l0) * att_lo + _bfr(l1) * att_hi), ev)
            exv = jnp.exp(ev - mvec)
            for i in range(16):
                ii = base + i
                exs = exv[i]
                r0 = exs * xlg[ii, pl.ds(0, 16)]
                r1 = exs * xlg[ii, pl.ds(16, 16)]
                r1 = jnp.where(lane == 4, exs, r1)
                vbuf[ii, pl.ds(0, 16)] = r0
                vbuf[ii, pl.ds(16, 16)] = r1
            return 0

        lax.fori_loop(0, 8, grp, 0)
        pltpu.sync_copy(vbuf, acc.at[dst_t.at[j]], add=True)
        return 0

    lax.fori_loop(0, KCH, chunk, 0)
    plsc.subcore_barrier()
    pltpu.sync_copy(acc.at[pl.ds(sid * ZR, ZR)],
                    out_h.at[pl.ds(cid * NACC + sid * ZR, ZR)])


@functools.lru_cache(maxsize=1)
def _build_decoder_edges():
  return functools.partial(
    pl.kernel,
    out_type=jax.ShapeDtypeStruct((NW, PWD), jnp.float32),
    mesh=_sc_mesh(),
    compiler_params=pltpu.CompilerParams(needs_layout_passes=False, use_tc_tiling_on_sc=False),
    scratch_types=[
        pltpu.VMEM((KD, 128), jnp.int32),
        pltpu.VMEM((KD, 128), jnp.int32),
        pltpu.VMEM((128, 64), jnp.float32),
        pltpu.VMEM((128, 64), jnp.float32),
        pltpu.VMEM((PWD,), jnp.float32),
        pltpu.VMEM((64,), jnp.float32),
        pltpu.VMEM((64,), jnp.float32),
        pltpu.VMEM_SHARED((N, 64), jnp.float32),
        pltpu.VMEM_SHARED((N, 64), jnp.float32),
        pltpu.SemaphoreType.DMA,
        pltpu.SemaphoreType.DMA,
    ],
  )(_decoder_edges_body)


def _decoder_edges(*args):
    return _build_decoder_edges()(*args)


def _decoder_edges_body(p_h, q_h, c_h, d_h, rows_h, cols_h, out_h,
                   rows_t, cols_t, pg, qg, zbuf, cbuf, dbuf, ps, qs,
                   sem1, sem2):
    cid = lax.axis_index("c")
    sid = lax.axis_index("s")
    wid = cid * NS + sid

    @pl.when(sid < NS - 1)
    def _():
        pltpu.sync_copy(p_h.at[pl.ds(sid * ZR, ZR)], ps.at[pl.ds(sid * ZR, ZR)])
        pltpu.sync_copy(q_h.at[pl.ds(sid * ZR, ZR)], qs.at[pl.ds(sid * ZR, ZR)])

    @pl.when(sid == NS - 1)
    def _():
        tail = N - (NS - 1) * ZR
        pltpu.sync_copy(p_h.at[pl.ds((NS - 1) * ZR, tail)],
                        ps.at[pl.ds((NS - 1) * ZR, tail)])
        pltpu.sync_copy(q_h.at[pl.ds((NS - 1) * ZR, tail)],
                        qs.at[pl.ds((NS - 1) * ZR, tail)])

    pltpu.sync_copy(rows_h.at[wid], rows_t)
    pltpu.sync_copy(cols_h.at[wid], cols_t)
    pltpu.sync_copy(c_h, cbuf)
    pltpu.sync_copy(d_h, dbuf)
    cks = [cbuf[pl.ds(16 * k, 16)] for k in range(4)]
    dks = [dbuf[pl.ds(16 * k, 16)] for k in range(4)]
    lane = lax.iota(jnp.int32, 16)
    plsc.subcore_barrier()

    def chunk(j, _):
        g1 = pltpu.async_copy(ps.at[rows_t.at[j]], pg, sem1)
        g2 = pltpu.async_copy(qs.at[cols_t.at[j]], qg, sem2)
        g1.wait()
        g2.wait()

        def grp(g, _):
            base = g * 16
            zv = jnp.zeros((16,), jnp.float32)
            for i in range(16):
                ii = base + i
                w = None
                for k in range(4):
                    h = _bfr(jnp.maximum(
                        pg[ii, pl.ds(16 * k, 16)] + qg[ii, pl.ds(16 * k, 16)]
                        + cks[k], 0.0))
                    w = h * dks[k] if w is None else w + h * dks[k]
                zv = jnp.where(lane == i, jnp.sum(w), zv)
            zbuf[pl.ds(j * 128 + base, 16)] = zv
            return 0

        lax.fori_loop(0, 8, grp, 0)
        return 0

    lax.fori_loop(0, KD, chunk, 0)
    neg = jnp.full((16,), -1e30, jnp.float32)
    for g in range((PWD - EDV) // 16):
        zbuf[pl.ds(EDV + g * 16, 16)] = neg
    pltpu.sync_copy(zbuf, out_h.at[wid])


def _gnn_edge_sim(xl, xr, attp, mv, src_p, dst_p):
    src = src_p.reshape(-1)
    dst = dst_p.reshape(-1)
    e = jax.nn.leaky_relu(xl[src] + xr[dst], negative_slope=0.2) @ attp
    ex = jnp.exp(e - mv[0])
    val = ex[:, None] * xl[src]
    val = val.at[:, 20].set(ex)
    wid = jnp.arange(src.shape[0]) // PW
    cid = wid // NS
    seg = cid * NACC + dst
    return jax.ops.segment_sum(val, seg, num_segments=2 * NACC)


def _dec_sim(P, Q, c0, d2, rows_p, cols_p):
    rows = rows_p.reshape(NW, -1)
    cols = cols_p.reshape(NW, -1)
    h = jax.nn.relu(P[rows] + Q[cols] + c0)
    z = h @ d2
    mask = jnp.arange(PWD)[None, :] < EDV
    return jnp.where(mask, z, -1e30)


def _pad_w(w):
    din = w.shape[0]
    dinp = HP if din <= HP else din
    out = jnp.zeros((dinp, HP), jnp.float32)
    return out.at[:din, :w.shape[1]].set(w)


def _pad_v(v):
    return jnp.zeros((1, HP), jnp.float32).at[0, :v.shape[0]].set(v)


def kernel(x, edge_index, node_id, Wl1, bl1, Wr1, br1, att1, b1,
           Wl2, bl2, Wr2, br2, att2, b2, Wl3, bl3, Wr3, br3, att3, b3,
           D1w, D1b, D2w, D2b):
    i32 = edge_index.dtype
    loops = jnp.arange(N, dtype=i32)
    src = jnp.concatenate([edge_index[0], loops])
    dst = jnp.concatenate([edge_index[1], loops])
    pad = EPAD - EP
    src_p = jnp.concatenate([src, jnp.zeros((pad,), i32)]).reshape(NW, KCH, 128)
    dst_p = jnp.concatenate([dst, jnp.full((pad,), N + 8, i32)]).reshape(NW, KCH, 128)

    rows_p = jnp.pad(edge_index[0].reshape(NW, EDV),
                     ((0, 0), (0, PWD - EDV))).reshape(NW, KD, 128)
    cols_p = jnp.pad(edge_index[1].reshape(NW, EDV),
                     ((0, 0), (0, PWD - EDV))).reshape(NW, KD, 128)

    wl = [_pad_w(Wl1), _pad_w(Wl2), _pad_w(Wl3)]
    wr = [_pad_w(Wr1), _pad_w(Wr2), _pad_w(Wr3)]
    blv = [_pad_v(bl1), _pad_v(bl2), _pad_v(bl3)]
    brv = [_pad_v(br1), _pad_v(br2), _pad_v(br3)]
    bf = lambda a: a.astype(jnp.bfloat16).astype(jnp.float32)
    attp = [jnp.pad(bf(att1), (0, HP - H)), jnp.pad(bf(att2), (0, HP - H)),
            jnp.pad(bf(att3), (0, HP - H))]
    bv = [_pad_v(b1), _pad_v(b2), _pad_v(b3)]

    def mshift(cm, ap):
        return jnp.full((16,), jnp.sum(jnp.abs(ap) * (cm[0] + cm[1])),
                        jnp.float32)

    xl, xr, cm = _proj_first(x, wl[0], blv[0], wr[0], brv[0])
    acc1 = _gnn_edge(xl, xr, attp[0], mshift(cm, attp[0]), src_p, dst_p)
    x1p, xl, xr, cm = _combine_proj(acc1, bv[0], wl[1], blv[1], wr[1], brv[1])
    acc2 = _gnn_edge(xl, xr, attp[1], mshift(cm, attp[1]), src_p, dst_p)
    x2p, xl, xr, cm = _combine_proj(acc2, bv[1], wl[2], blv[2], wr[2], brv[2])
    acc3 = _gnn_edge(xl, xr, attp[2], mshift(cm, attp[2]), src_p, dst_p)

    dpad = jnp.zeros((HP, 64), jnp.float32)
    a1 = dpad.at[:H].set(D1w[0:20])
    a2 = dpad.at[:H].set(D1w[20:40])
    a3 = dpad.at[:H].set(D1w[40:60])
    c1 = dpad.at[:H].set(D1w[60:80])
    c2 = dpad.at[:H].set(D1w[80:100])
    c3 = dpad.at[:H].set(D1w[100:120])
    x3p, P, Q = _final_combine(acc3, bv[2], x1p, x2p, a1, a2, a3, c1, c2, c3)

    c0 = (jnp.dot(bf(x1p[node_id, :H]), bf(D1w[120:140]), precision=_HIGH)
          + jnp.dot(bf(x2p[node_id, :H]), bf(D1w[140:160]), precision=_HIGH)
          + jnp.dot(bf(x3p[node_id, :H]), bf(D1w[160:180]), precision=_HIGH)
          + D1b)
    z = _decoder_edges(P, Q, c0, bf(D2w[:, 0]), rows_p, cols_p)
    outp = _sparsemax(z)
    return outp[:, :EDV].reshape(E, 1)


# speedup vs baseline: 27.5189x; 1.0553x over previous
import functools

import jax
import jax.numpy as jnp
from jax import lax
from jax.experimental import pallas as pl
from jax.experimental.pallas import tpu as pltpu
from jax.experimental.pallas import tpu_sc as plsc

N = 10000
E = 320000
H = 20
HP = 32
NACC = N + 112
NC, NS = 2, 16
NW = NC * NS
ZR = NACC // NS

EP = E + N
KCH = 81
PW = KCH * 128
EPAD = NW * PW

EDV = E // NW
KD = 79
PWD = KD * 128

_HIGH = jax.lax.Precision.HIGHEST


@functools.lru_cache(maxsize=1)
def _sc_mesh():
    return plsc.VectorSubcoreMesh(core_axis_name="c", subcore_axis_name="s",
                                  num_cores=NC, num_subcores=NS)


def _dot(a, b):
    return jnp.dot(a.astype(jnp.bfloat16), b.astype(jnp.bfloat16),
                   preferred_element_type=jnp.float32)


def _proj_first(x, wlp, blp, wrp, brp):
    def body(x_ref, wl_ref, bl_ref, wr_ref, br_ref, xl_ref, xr_ref, cm_ref):
        xv = x_ref[...]
        xl = _dot(xv, wl_ref[...]) + bl_ref[...]
        xr = _dot(xv, wr_ref[...]) + br_ref[...]
        xl_ref[...] = xl
        xr_ref[...] = xr
        ml = jnp.max(jnp.abs(xl), axis=0, keepdims=True)
        mr = jnp.max(jnp.abs(xr), axis=0, keepdims=True)
        cm_ref[...] = jnp.concatenate(
            [ml, mr, jnp.zeros((6, HP), jnp.float32)], axis=0)

    return pl.pallas_call(
        body,
        out_shape=[
            jax.ShapeDtypeStruct((N, HP), jnp.float32),
            jax.ShapeDtypeStruct((N, HP), jnp.float32),
            jax.ShapeDtypeStruct((8, HP), jnp.float32),
        ],
    )(x, wlp, blp, wrp, brp)


def _combine_proj(acc, bp, wlp, blp, wrp, brp):
    def body(acc_ref, b_ref, wl_ref, bl_ref, wr_ref, br_ref,
             x_ref, xl_ref, xr_ref, cm_ref):
        a = acc_ref[0:N, :] + acc_ref[NACC:NACC + N, :]
        s = a[:, 20:21]
        xn = jnp.maximum(a / s + b_ref[...], 0.0)
        x_ref[...] = xn
        xl = _dot(xn, wl_ref[...]) + bl_ref[...]
        xr = _dot(xn, wr_ref[...]) + br_ref[...]
        xl_ref[...] = xl
        xr_ref[...] = xr
        ml = jnp.max(jnp.abs(xl), axis=0, keepdims=True)
        mr = jnp.max(jnp.abs(xr), axis=0, keepdims=True)
        cm_ref[...] = jnp.concatenate(
            [ml, mr, jnp.zeros((6, HP), jnp.float32)], axis=0)

    return pl.pallas_call(
        body,
        out_shape=[
            jax.ShapeDtypeStruct((N, HP), jnp.float32),
            jax.ShapeDtypeStruct((N, HP), jnp.float32),
            jax.ShapeDtypeStruct((N, HP), jnp.float32),
            jax.ShapeDtypeStruct((8, HP), jnp.float32),
        ],
    )(acc, bp, wlp, blp, wrp, brp)


def _final_combine(acc, bp, x1p, x2p, a1, a2, a3, b1, b2, b3):
    BLK = 2000

    def body(accA_ref, accB_ref, b_ref, x1_ref, x2_ref,
             a1_ref, a2_ref, a3_ref, b1_ref, b2_ref, b3_ref,
             x3_ref, p_ref, q_ref):
        a = accA_ref[0] + accB_ref[0]
        s = a[:, 20:21]
        x3 = jnp.maximum(a / s + b_ref[...], 0.0)
        x3_ref[...] = x3
        x1 = x1_ref[...]
        x2 = x2_ref[...]
        p_ref[...] = _dot(x1, a1_ref[...]) + _dot(x2, a2_ref[...]) + _dot(x3, a3_ref[...])
        q_ref[...] = _dot(x1, b1_ref[...]) + _dot(x2, b2_ref[...]) + _dot(x3, b3_ref[...])

    acc2 = acc.reshape(2, NACC, HP)
    full = lambda shp: pl.BlockSpec(shp, lambda i: tuple(0 for _ in shp))
    return pl.pallas_call(
        body,
        grid=(N // BLK,),
        in_specs=[
            pl.BlockSpec((1, BLK, HP), lambda i: (0, i, 0)),
            pl.BlockSpec((1, BLK, HP), lambda i: (1, i, 0)),
            full((1, HP)),
            pl.BlockSpec((BLK, HP), lambda i: (i, 0)),
            pl.BlockSpec((BLK, HP), lambda i: (i, 0)),
            full((HP, 64)), full((HP, 64)), full((HP, 64)),
            full((HP, 64)), full((HP, 64)), full((HP, 64)),
        ],
        out_specs=[
            pl.BlockSpec((BLK, HP), lambda i: (i, 0)),
            pl.BlockSpec((BLK, 64), lambda i: (i, 0)),
            pl.BlockSpec((BLK, 64), lambda i: (i, 0)),
        ],
        out_shape=[
            jax.ShapeDtypeStruct((N, HP), jnp.float32),
            jax.ShapeDtypeStruct((N, 64), jnp.float32),
            jax.ShapeDtypeStruct((N, 64), jnp.float32),
        ],
    )(acc2, acc2, bp, x1p, x2p, a1, a2, a3, b1, b2, b3)


def _sparsemax(z):
    def body(z_ref, o_ref):
        zv = z_ref[...]
        zmax = jnp.max(zv)

        def bis(_, lh):
            lo, hi = lh
            mid = 0.5 * (lo + hi)
            f = jnp.sum(jnp.maximum(zv - mid, 0.0)) - 1.0
            good = f > 0.0
            return jnp.where(good, mid, lo), jnp.where(good, hi, mid)

        lo, hi = lax.fori_loop(0, 40, bis, (zmax - 1.0, zmax))
        ta = 0.5 * (lo + hi)
        sup = zv > ta
        k = jnp.sum(sup.astype(jnp.float32))
        tau = (jnp.sum(jnp.where(sup, zv, 0.0)) - 1.0) / k
        o_ref[...] = jnp.maximum(zv - tau, 0.0)

    return pl.pallas_call(
        body,
        out_shape=jax.ShapeDtypeStruct((NW, PWD), jnp.float32),
    )(z)


def _bfr(v):
    u = plsc.bitcast(v, jnp.uint32)
    r = (u + jnp.uint32(0x7FFF) + ((u >> jnp.uint32(16)) & jnp.uint32(1))) \
        & jnp.uint32(0xFFFF0000)
    return plsc.bitcast(r, jnp.float32)


@functools.lru_cache(maxsize=1)
def _build_gnn_edge():
  return functools.partial(
    pl.kernel,
    out_type=jax.ShapeDtypeStruct((2 * NACC, HP), jnp.float32),
    mesh=_sc_mesh(),
    compiler_params=pltpu.CompilerParams(needs_layout_passes=False, use_tc_tiling_on_sc=False),
    scratch_types=[
        pltpu.VMEM((KCH, 128), jnp.int32),
        pltpu.VMEM((KCH, 128), jnp.int32),
        pltpu.VMEM((128, HP), jnp.float32),
        pltpu.VMEM((128, HP), jnp.float32),
        pltpu.VMEM((128, HP), jnp.float32),
        pltpu.VMEM((HP,), jnp.float32),
        pltpu.VMEM((16,), jnp.float32),
        pltpu.VMEM((ZR, HP), jnp.float32),
        pltpu.VMEM_SHARED((NACC, HP), jnp.float32),
        pltpu.VMEM_SHARED((N, HP), jnp.float32),
        pltpu.VMEM_SHARED((N, HP), jnp.float32),
        pltpu.SemaphoreType.DMA,
        pltpu.SemaphoreType.DMA,
    ],
  )(_gnn_edge_body)


def _gnn_edge(*args):
    return _build_gnn_edge()(*args)


def _gnn_edge_body(xl_h, xr_h, att_h, mv_h, src_h, dst_h, out_h,
              src_t, dst_t, xlg, xrg, vbuf, attv, mvr, stage, acc, xls, xrs,
              sem1, sem2):
    cid = lax.axis_index("c")
    sid = lax.axis_index("s")
    wid = cid * NS + sid

    zer = jnp.zeros((16,), jnp.float32)

    def zbody(i, _):
        stage[i, pl.ds(0, 16)] = zer
        stage[i, pl.ds(16, 16)] = zer
        return 0

    lax.fori_loop(0, ZR, zbody, 0)
    pltpu.sync_copy(stage, acc.at[pl.ds(sid * ZR, ZR)])

    @pl.when(sid < NS - 1)
    def _():
        pltpu.sync_copy(xl_h.at[pl.ds(sid * ZR, ZR)], xls.at[pl.ds(sid * ZR, ZR)])
        pltpu.sync_copy(xr_h.at[pl.ds(sid * ZR, ZR)], xrs.at[pl.ds(sid * ZR, ZR)])

    @pl.when(sid == NS - 1)
    def _():
        tail = N - (NS - 1) * ZR
        pltpu.sync_copy(xl_h.at[pl.ds((NS - 1) * ZR, tail)],
                        xls.at[pl.ds((NS - 1) * ZR, tail)])
        pltpu.sync_copy(xr_h.at[pl.ds((NS - 1) * ZR, tail)],
                        xrs.at[pl.ds((NS - 1) * ZR, tail)])

    pltpu.sync_copy(src_h.at[wid], src_t)
    pltpu.sync_copy(dst_h.at[wid], dst_t)
    pltpu.sync_copy(att_h, attv)
    pltpu.sync_copy(mv_h, mvr)
    plsc.subcore_barrier()

    att_lo = attv[pl.ds(0, 16)]
    att_hi = attv[pl.ds(16, 16)]
    mvec = mvr[...]
    lane = lax.iota(jnp.int32, 16)

    def chunk(j, _):
        g1 = pltpu.async_copy(xls.at[src_t.at[j]], xlg, sem1)
        g2 = pltpu.async_copy(xrs.at[dst_t.at[j]], xrg, sem2)
        g1.wait()
        g2.wait()

        def grp(g, _):
            base = g * 16
            ev = jnp.zeros((16,), jnp.float32)
            for i in range(16):
                ii = base + i
                t0 = xlg[ii, pl.ds(0, 16)] + xrg[ii, pl.ds(0, 16)]
                t1 = xlg[ii, pl.ds(16, 16)] + xrg[ii, pl.ds(16, 16)]
                l0 = jnp.maximum(t0, 0.2 * t0)
                l1 = jnp.maximum(t1, 0.2 * t1)
                ev = jnp.where(
                    lane == i,
                    jnp.sum(_bfr(l0) * att_lo + _bfr(l1) * att_hi), ev)
            exv = jnp.exp(ev - mvec)
            for i in range(16):
                ii = base + i
                exs = exv[i]
                r0 = exs * xlg[ii, pl.ds(0, 16)]
                r1 = exs * xlg[ii, pl.ds(16, 16)]
                r1 = jnp.where(lane == 4, exs, r1)
                vbuf[ii, pl.ds(0, 16)] = r0
                vbuf[ii, pl.ds(16, 16)] = r1
            return 0

        lax.fori_loop(0, 8, grp, 0)
        pltpu.sync_copy(vbuf, acc.at[dst_t.at[j]], add=True)
        return 0

    lax.fori_loop(0, KCH, chunk, 0)
    plsc.subcore_barrier()
    pltpu.sync_copy(acc.at[pl.ds(sid * ZR, ZR)],
                    out_h.at[pl.ds(cid * NACC + sid * ZR, ZR)])


@functools.lru_cache(maxsize=1)
def _build_decoder_edges():
  return functools.partial(
    pl.kernel,
    out_type=jax.ShapeDtypeStruct((NW, PWD), jnp.float32),
    mesh=_sc_mesh(),
    compiler_params=pltpu.CompilerParams(needs_layout_passes=False, use_tc_tiling_on_sc=False),
    scratch_types=[
        pltpu.VMEM((KD, 128), jnp.int32),
        pltpu.VMEM((KD, 128), jnp.int32),
        pltpu.VMEM((128, 64), jnp.float32),
        pltpu.VMEM((128, 64), jnp.float32),
        pltpu.VMEM((PWD,), jnp.float32),
        pltpu.VMEM((64,), jnp.float32),
        pltpu.VMEM((64,), jnp.float32),
        pltpu.VMEM_SHARED((N, 64), jnp.float32),
        pltpu.VMEM_SHARED((N, 64), jnp.float32),
        pltpu.SemaphoreType.DMA,
        pltpu.SemaphoreType.DMA,
    ],
  )(_decoder_edges_body)


def _decoder_edges(*args):
    return _build_decoder_edges()(*args)


def _decoder_edges_body(p_h, q_h, c_h, d_h, rows_h, cols_h, out_h,
                   rows_t, cols_t, pg, qg, zbuf, cbuf, dbuf, ps, qs,
                   sem1, sem2):
    cid = lax.axis_index("c")
    sid = lax.axis_index("s")
    wid = cid * NS + sid

    @pl.when(sid < NS - 1)
    def _():
        pltpu.sync_copy(p_h.at[pl.ds(sid * ZR, ZR)], ps.at[pl.ds(sid * ZR, ZR)])
        pltpu.sync_copy(q_h.at[pl.ds(sid * ZR, ZR)], qs.at[pl.ds(sid * ZR, ZR)])

    @pl.when(sid == NS - 1)
    def _():
        tail = N - (NS - 1) * ZR
        pltpu.sync_copy(p_h.at[pl.ds((NS - 1) * ZR, tail)],
                        ps.at[pl.ds((NS - 1) * ZR, tail)])
        pltpu.sync_copy(q_h.at[pl.ds((NS - 1) * ZR, tail)],
                        qs.at[pl.ds((NS - 1) * ZR, tail)])

    pltpu.sync_copy(rows_h.at[wid], rows_t)
    pltpu.sync_copy(cols_h.at[wid], cols_t)
    pltpu.sync_copy(c_h, cbuf)
    pltpu.sync_copy(d_h, dbuf)
    cks = [cbuf[pl.ds(16 * k, 16)] for k in range(4)]
    dks = [dbuf[pl.ds(16 * k, 16)] for k in range(4)]
    lane = lax.iota(jnp.int32, 16)
    plsc.subcore_barrier()

    def chunk(j, _):
        g1 = pltpu.async_copy(ps.at[rows_t.at[j]], pg, sem1)
        g2 = pltpu.async_copy(qs.at[cols_t.at[j]], qg, sem2)
        g1.wait()
        g2.wait()

        def grp(g, _):
            base = g * 16
            zv = jnp.zeros((16,), jnp.float32)
            for i in range(16):
                ii = base + i
                w = None
                for k in range(4):
                    h = _bfr(jnp.maximum(
                        pg[ii, pl.ds(16 * k, 16)] + qg[ii, pl.ds(16 * k, 16)]
                        + cks[k], 0.0))
                    w = h * dks[k] if w is None else w + h * dks[k]
                zv = jnp.where(lane == i, jnp.sum(w), zv)
            zbuf[pl.ds(j * 128 + base, 16)] = zv
            return 0

        lax.fori_loop(0, 8, grp, 0)
        return 0

    lax.fori_loop(0, KD, chunk, 0)
    neg = jnp.full((16,), -1e30, jnp.float32)
    for g in range((PWD - EDV) // 16):
        zbuf[pl.ds(EDV + g * 16, 16)] = neg
    pltpu.sync_copy(zbuf, out_h.at[wid])


def _gnn_edge_sim(xl, xr, attp, mv, src_p, dst_p):
    src = src_p.reshape(-1)
    dst = dst_p.reshape(-1)
    e = jax.nn.leaky_relu(xl[src] + xr[dst], negative_slope=0.2) @ attp
    ex = jnp.exp(e - mv[0])
    val = ex[:, None] * xl[src]
    val = val.at[:, 20].set(ex)
    wid = jnp.arange(src.shape[0]) // PW
    cid = wid // NS
    seg = cid * NACC + dst
    return jax.ops.segment_sum(val, seg, num_segments=2 * NACC)


def _dec_sim(P, Q, c0, d2, rows_p, cols_p):
    rows = rows_p.reshape(NW, -1)
    cols = cols_p.reshape(NW, -1)
    h = jax.nn.relu(P[rows] + Q[cols] + c0)
    z = h @ d2
    mask = jnp.arange(PWD)[None, :] < EDV
    return jnp.where(mask, z, -1e30)


def _pad_w(w):
    din = w.shape[0]
    dinp = HP if din <= HP else din
    out = jnp.zeros((dinp, HP), jnp.float32)
    return out.at[:din, :w.shape[1]].set(w)


def _pad_v(v):
    return jnp.zeros((1, HP), jnp.float32).at[0, :v.shape[0]].set(v)


def kernel(x, edge_index, node_id, Wl1, bl1, Wr1, br1, att1, b1,
           Wl2, bl2, Wr2, br2, att2, b2, Wl3, bl3, Wr3, br3, att3, b3,
           D1w, D1b, D2w, D2b):
    i32 = edge_index.dtype
    loops = jnp.arange(N, dtype=i32)
    src = jnp.concatenate([edge_index[0], loops])
    dst = jnp.concatenate([edge_index[1], loops])
    pad = EPAD - EP
    src_p = jnp.concatenate([src, jnp.zeros((pad,), i32)]).reshape(NW, KCH, 128)
    dst_p = jnp.concatenate([dst, jnp.full((pad,), N + 8, i32)]).reshape(NW, KCH, 128)

    rows_p = jnp.pad(edge_index[0].reshape(NW, EDV),
                     ((0, 0), (0, PWD - EDV))).reshape(NW, KD, 128)
    cols_p = jnp.pad(edge_index[1].reshape(NW, EDV),
                     ((0, 0), (0, PWD - EDV))).reshape(NW, KD, 128)

    wl = [_pad_w(Wl1), _pad_w(Wl2), _pad_w(Wl3)]
    wr = [_pad_w(Wr1), _pad_w(Wr2), _pad_w(Wr3)]
    blv = [_pad_v(bl1), _pad_v(bl2), _pad_v(bl3)]
    brv = [_pad_v(br1), _pad_v(br2), _pad_v(br3)]
    bf = lambda a: a.astype(jnp.bfloat16).astype(jnp.float32)
    attp = [jnp.pad(bf(att1), (0, HP - H)), jnp.pad(bf(att2), (0, HP - H)),
            jnp.pad(bf(att3), (0, HP - H))]
    bv = [_pad_v(b1), _pad_v(b2), _pad_v(b3)]

    def mshift(cm, ap):
        return jnp.full((16,), jnp.sum(jnp.abs(ap) * (cm[0] + cm[1])),
                        jnp.float32)

    xl, xr, cm = _proj_first(x, wl[0], blv[0], wr[0], brv[0])
    acc1 = _gnn_edge(xl, xr, attp[0], mshift(cm, attp[0]), src_p, dst_p)
    x1p, xl, xr, cm = _combine_proj(acc1, bv[0], wl[1], blv[1], wr[1], brv[1])
    acc2 = _gnn_edge(xl, xr, attp[1], mshift(cm, attp[1]), src_p, dst_p)
    x2p, xl, xr, cm = _combine_proj(acc2, bv[1], wl[2], blv[2], wr[2], brv[2])
    acc3 = _gnn_edge(xl, xr, attp[2], mshift(cm, attp[2]), src_p, dst_p)

    dpad = jnp.zeros((HP, 64), jnp.float32)
    a1 = dpad.at[:H].set(D1w[0:20])
    a2 = dpad.at[:H].set(D1w[20:40])
    a3 = dpad.at[:H].set(D1w[40:60])
    c1 = dpad.at[:H].set(D1w[60:80])
    c2 = dpad.at[:H].set(D1w[80:100])
    c3 = dpad.at[:H].set(D1w[100:120])
    x3p, P, Q = _final_combine(acc3, bv[2], x1p, x2p, a1, a2, a3, c1, c2, c3)

    c0 = (jnp.dot(bf(x1p[node_id, :H]), bf(D1w[120:140]), precision=_HIGH)
          + jnp.dot(bf(x2p[node_id, :H]), bf(D1w[140:160]), precision=_HIGH)
          + jnp.dot(bf(x3p[node_id, :H]), bf(D1w[160:180]), precision=_HIGH)
          + D1b)
    z = _decoder_edges(P, Q, c0, bf(D2w[:, 0]), rows_p, cols_p)
    outp = _sparsemax(z)
    return outp[:, :EDV].reshape(E, 1)


# speedup vs baseline: 28.8773x; 1.0494x over previous
import functools

import jax
import jax.numpy as jnp
from jax import lax
from jax.experimental import pallas as pl
from jax.experimental.pallas import tpu as pltpu
from jax.experimental.pallas import tpu_sc as plsc

N = 10000
E = 320000
H = 20
HP = 32
NACC = N + 112
NC, NS = 2, 16
NW = NC * NS
ZR = NACC // NS

EP = E + N
KCH = 81
PW = KCH * 128
EPAD = NW * PW

EDV = E // NW
KD = 79
PWD = KD * 128

_HIGH = jax.lax.Precision.HIGHEST


@functools.lru_cache(maxsize=1)
def _sc_mesh():
    return plsc.VectorSubcoreMesh(core_axis_name="c", subcore_axis_name="s",
                                  num_cores=NC, num_subcores=NS)


def _dot(a, b):
    return jnp.dot(a.astype(jnp.bfloat16), b.astype(jnp.bfloat16),
                   preferred_element_type=jnp.float32)


def _proj_first(x, wlp, blp, wrp, brp):
    def body(x_ref, wl_ref, bl_ref, wr_ref, br_ref, xl_ref, xr_ref, cm_ref):
        xv = x_ref[...]
        xl = _dot(xv, wl_ref[...]) + bl_ref[...]
        xr = _dot(xv, wr_ref[...]) + br_ref[...]
        col = lax.broadcasted_iota(jnp.int32, (N, HP), 1)
        xl_ref[...] = jnp.where(col == 20, 1.0, xl)
        xr_ref[...] = xr
        ml = jnp.max(jnp.abs(xl), axis=0, keepdims=True)
        mr = jnp.max(jnp.abs(xr), axis=0, keepdims=True)
        cm_ref[...] = jnp.concatenate(
            [ml, mr, jnp.zeros((6, HP), jnp.float32)], axis=0)

    return pl.pallas_call(
        body,
        out_shape=[
            jax.ShapeDtypeStruct((N, HP), jnp.float32),
            jax.ShapeDtypeStruct((N, HP), jnp.float32),
            jax.ShapeDtypeStruct((8, HP), jnp.float32),
        ],
    )(x, wlp, blp, wrp, brp)


def _combine_proj(acc, bp, wlp, blp, wrp, brp):
    def body(acc_ref, b_ref, wl_ref, bl_ref, wr_ref, br_ref,
             x_ref, xl_ref, xr_ref, cm_ref):
        a = acc_ref[0:N, :] + acc_ref[NACC:NACC + N, :]
        s = a[:, 20:21]
        xn = jnp.maximum(a / s + b_ref[...], 0.0)
        x_ref[...] = xn
        xl = _dot(xn, wl_ref[...]) + bl_ref[...]
        xr = _dot(xn, wr_ref[...]) + br_ref[...]
        col = lax.broadcasted_iota(jnp.int32, (N, HP), 1)
        xl_ref[...] = jnp.where(col == 20, 1.0, xl)
        xr_ref[...] = xr
        ml = jnp.max(jnp.abs(xl), axis=0, keepdims=True)
        mr = jnp.max(jnp.abs(xr), axis=0, keepdims=True)
        cm_ref[...] = jnp.concatenate(
            [ml, mr, jnp.zeros((6, HP), jnp.float32)], axis=0)

    return pl.pallas_call(
        body,
        out_shape=[
            jax.ShapeDtypeStruct((N, HP), jnp.float32),
            jax.ShapeDtypeStruct((N, HP), jnp.float32),
            jax.ShapeDtypeStruct((N, HP), jnp.float32),
            jax.ShapeDtypeStruct((8, HP), jnp.float32),
        ],
    )(acc, bp, wlp, blp, wrp, brp)


def _final_combine(acc, bp, x1p, x2p, a1, a2, a3, b1, b2, b3):
    BLK = 2000

    def body(accA_ref, accB_ref, b_ref, x1_ref, x2_ref,
             a1_ref, a2_ref, a3_ref, b1_ref, b2_ref, b3_ref,
             x3_ref, p_ref, q_ref):
        a = accA_ref[0] + accB_ref[0]
        s = a[:, 20:21]
        x3 = jnp.maximum(a / s + b_ref[...], 0.0)
        x3_ref[...] = x3
        x1 = x1_ref[...]
        x2 = x2_ref[...]
        p_ref[...] = _dot(x1, a1_ref[...]) + _dot(x2, a2_ref[...]) + _dot(x3, a3_ref[...])
        q_ref[...] = _dot(x1, b1_ref[...]) + _dot(x2, b2_ref[...]) + _dot(x3, b3_ref[...])

    acc2 = acc.reshape(2, NACC, HP)
    full = lambda shp: pl.BlockSpec(shp, lambda i: tuple(0 for _ in shp))
    return pl.pallas_call(
        body,
        grid=(N // BLK,),
        in_specs=[
            pl.BlockSpec((1, BLK, HP), lambda i: (0, i, 0)),
            pl.BlockSpec((1, BLK, HP), lambda i: (1, i, 0)),
            full((1, HP)),
            pl.BlockSpec((BLK, HP), lambda i: (i, 0)),
            pl.BlockSpec((BLK, HP), lambda i: (i, 0)),
            full((HP, 64)), full((HP, 64)), full((HP, 64)),
            full((HP, 64)), full((HP, 64)), full((HP, 64)),
        ],
        out_specs=[
            pl.BlockSpec((BLK, HP), lambda i: (i, 0)),
            pl.BlockSpec((BLK, 64), lambda i: (i, 0)),
            pl.BlockSpec((BLK, 64), lambda i: (i, 0)),
        ],
        out_shape=[
            jax.ShapeDtypeStruct((N, HP), jnp.float32),
            jax.ShapeDtypeStruct((N, 64), jnp.float32),
            jax.ShapeDtypeStruct((N, 64), jnp.float32),
        ],
    )(acc2, acc2, bp, x1p, x2p, a1, a2, a3, b1, b2, b3)


def _sparsemax(z):
    def body(z_ref, o_ref):
        zv = z_ref[...]
        zmax = jnp.max(zv)

        def bis(_, lh):
            lo, hi = lh
            mid = 0.5 * (lo + hi)
            f = jnp.sum(jnp.maximum(zv - mid, 0.0)) - 1.0
            good = f > 0.0
            return jnp.where(good, mid, lo), jnp.where(good, hi, mid)

        lo, hi = lax.fori_loop(0, 40, bis, (zmax - 1.0, zmax))
        ta = 0.5 * (lo + hi)
        sup = zv > ta
        k = jnp.sum(sup.astype(jnp.float32))
        tau = (jnp.sum(jnp.where(sup, zv, 0.0)) - 1.0) / k
        o_ref[...] = jnp.maximum(zv - tau, 0.0)

    return pl.pallas_call(
        body,
        out_shape=jax.ShapeDtypeStruct((NW, PWD), jnp.float32),
    )(z)


def _bfr(v):
    u = plsc.bitcast(v, jnp.uint32)
    r = (u + jnp.uint32(0x7FFF) + ((u >> jnp.uint32(16)) & jnp.uint32(1))) \
        & jnp.uint32(0xFFFF0000)
    return plsc.bitcast(r, jnp.float32)


@functools.lru_cache(maxsize=1)
def _build_gnn_edge():
  return functools.partial(
    pl.kernel,
    out_type=jax.ShapeDtypeStruct((2 * NACC, HP), jnp.float32),
    mesh=_sc_mesh(),
    compiler_params=pltpu.CompilerParams(needs_layout_passes=False, use_tc_tiling_on_sc=False),
    scratch_types=[
        pltpu.VMEM((KCH, 128), jnp.int32),
        pltpu.VMEM((KCH, 128), jnp.int32),
        pltpu.VMEM((128, HP), jnp.float32),
        pltpu.VMEM((128, HP), jnp.float32),
        pltpu.VMEM((128, HP), jnp.float32),
        pltpu.VMEM((HP,), jnp.float32),
        pltpu.VMEM((16,), jnp.float32),
        pltpu.VMEM((ZR, HP), jnp.float32),
        pltpu.VMEM_SHARED((NACC, HP), jnp.float32),
        pltpu.VMEM_SHARED((N, HP), jnp.float32),
        pltpu.VMEM_SHARED((N, HP), jnp.float32),
        pltpu.SemaphoreType.DMA,
        pltpu.SemaphoreType.DMA,
    ],
  )(_gnn_edge_body)


def _gnn_edge(*args):
    return _build_gnn_edge()(*args)


def _gnn_edge_body(xl_h, xr_h, att_h, mv_h, src_h, dst_h, out_h,
              src_t, dst_t, xlg, xrg, vbuf, attv, mvr, stage, acc, xls, xrs,
              sem1, sem2):
    cid = lax.axis_index("c")
    sid = lax.axis_index("s")
    wid = cid * NS + sid

    zer = jnp.zeros((16,), jnp.float32)

    def zbody(i, _):
        stage[i, pl.ds(0, 16)] = zer
        stage[i, pl.ds(16, 16)] = zer
        return 0

    lax.fori_loop(0, ZR, zbody, 0)
    pltpu.sync_copy(stage, acc.at[pl.ds(sid * ZR, ZR)])

    @pl.when(sid < NS - 1)
    def _():
        pltpu.sync_copy(xl_h.at[pl.ds(sid * ZR, ZR)], xls.at[pl.ds(sid * ZR, ZR)])
        pltpu.sync_copy(xr_h.at[pl.ds(sid * ZR, ZR)], xrs.at[pl.ds(sid * ZR, ZR)])

    @pl.when(sid == NS - 1)
    def _():
        tail = N - (NS - 1) * ZR
        pltpu.sync_copy(xl_h.at[pl.ds((NS - 1) * ZR, tail)],
                        xls.at[pl.ds((NS - 1) * ZR, tail)])
        pltpu.sync_copy(xr_h.at[pl.ds((NS - 1) * ZR, tail)],
                        xrs.at[pl.ds((NS - 1) * ZR, tail)])

    pltpu.sync_copy(src_h.at[wid], src_t)
    pltpu.sync_copy(dst_h.at[wid], dst_t)
    pltpu.sync_copy(att_h, attv)
    pltpu.sync_copy(mv_h, mvr)
    plsc.subcore_barrier()

    att_lo = attv[pl.ds(0, 16)]
    att_hi = attv[pl.ds(16, 16)]
    mvec = mvr[...]
    lane = lax.iota(jnp.int32, 16)

    def chunk(j, _):
        g1 = pltpu.async_copy(xls.at[src_t.at[j]], xlg, sem1)
        g2 = pltpu.async_copy(xrs.at[dst_t.at[j]], xrg, sem2)
        g1.wait()
        g2.wait()

        def grp(g, _):
            base = g * 16
            ev = jnp.zeros((16,), jnp.float32)
            for i in range(16):
                ii = base + i
                t0 = xlg[ii, pl.ds(0, 16)] + xrg[ii, pl.ds(0, 16)]
                t1 = xlg[ii, pl.ds(16, 16)] + xrg[ii, pl.ds(16, 16)]
                l0 = jnp.maximum(t0, 0.2 * t0)
                l1 = jnp.maximum(t1, 0.2 * t1)
                ev = jnp.where(
                    lane == i,
                    jnp.sum(_bfr(l0) * att_lo + _bfr(l1) * att_hi), ev)
            exv = jnp.exp(ev - mvec)
            for i in range(16):
                ii = base + i
                exs = exv[i]
                vbuf[ii, pl.ds(0, 16)] = exs * xlg[ii, pl.ds(0, 16)]
                vbuf[ii, pl.ds(16, 16)] = exs * xlg[ii, pl.ds(16, 16)]
            return 0

        lax.fori_loop(0, 8, grp, 0)
        pltpu.sync_copy(vbuf, acc.at[dst_t.at[j]], add=True)
        return 0

    lax.fori_loop(0, KCH, chunk, 0)
    plsc.subcore_barrier()
    pltpu.sync_copy(acc.at[pl.ds(sid * ZR, ZR)],
                    out_h.at[pl.ds(cid * NACC + sid * ZR, ZR)])


@functools.lru_cache(maxsize=1)
def _build_decoder_edges():
  return functools.partial(
    pl.kernel,
    out_type=jax.ShapeDtypeStruct((NW, PWD), jnp.float32),
    mesh=_sc_mesh(),
    compiler_params=pltpu.CompilerParams(needs_layout_passes=False, use_tc_tiling_on_sc=False),
    scratch_types=[
        pltpu.VMEM((KD, 128), jnp.int32),
        pltpu.VMEM((KD, 128), jnp.int32),
        pltpu.VMEM((128, 64), jnp.float32),
        pltpu.VMEM((128, 64), jnp.float32),
        pltpu.VMEM((PWD,), jnp.float32),
        pltpu.VMEM((64,), jnp.float32),
        pltpu.VMEM((64,), jnp.float32),
        pltpu.VMEM_SHARED((N, 64), jnp.float32),
        pltpu.VMEM_SHARED((N, 64), jnp.float32),
        pltpu.SemaphoreType.DMA,
        pltpu.SemaphoreType.DMA,
    ],
  )(_decoder_edges_body)


def _decoder_edges(*args):
    return _build_decoder_edges()(*args)


def _decoder_edges_body(p_h, q_h, c_h, d_h, rows_h, cols_h, out_h,
                   rows_t, cols_t, pg, qg, zbuf, cbuf, dbuf, ps, qs,
                   sem1, sem2):
    cid = lax.axis_index("c")
    sid = lax.axis_index("s")
    wid = cid * NS + sid

    @pl.when(sid < NS - 1)
    def _():
        pltpu.sync_copy(p_h.at[pl.ds(sid * ZR, ZR)], ps.at[pl.ds(sid * ZR, ZR)])
        pltpu.sync_copy(q_h.at[pl.ds(sid * ZR, ZR)], qs.at[pl.ds(sid * ZR, ZR)])

    @pl.when(sid == NS - 1)
    def _():
        tail = N - (NS - 1) * ZR
        pltpu.sync_copy(p_h.at[pl.ds((NS - 1) * ZR, tail)],
                        ps.at[pl.ds((NS - 1) * ZR, tail)])
        pltpu.sync_copy(q_h.at[pl.ds((NS - 1) * ZR, tail)],
                        qs.at[pl.ds((NS - 1) * ZR, tail)])

    pltpu.sync_copy(rows_h.at[wid], rows_t)
    pltpu.sync_copy(cols_h.at[wid], cols_t)
    pltpu.sync_copy(c_h, cbuf)
    pltpu.sync_copy(d_h, dbuf)
    cks = [cbuf[pl.ds(16 * k, 16)] for k in range(4)]
    dks = [dbuf[pl.ds(16 * k, 16)] for k in range(4)]
    lane = lax.iota(jnp.int32, 16)
    plsc.subcore_barrier()

    def chunk(j, _):
        g1 = pltpu.async_copy(ps.at[rows_t.at[j]], pg, sem1)
        g2 = pltpu.async_copy(qs.at[cols_t.at[j]], qg, sem2)
        g1.wait()
        g2.wait()

        def grp(g, _):
            base = g * 16
            zv = jnp.zeros((16,), jnp.float32)
            for i in range(16):
                ii = base + i
                w = None
                for k in range(4):
                    h = _bfr(jnp.maximum(
                        pg[ii, pl.ds(16 * k, 16)] + qg[ii, pl.ds(16 * k, 16)]
                        + cks[k], 0.0))
                    w = h * dks[k] if w is None else w + h * dks[k]
                zv = jnp.where(lane == i, jnp.sum(w), zv)
            zbuf[pl.ds(j * 128 + base, 16)] = zv
            return 0

        lax.fori_loop(0, 8, grp, 0)
        return 0

    lax.fori_loop(0, KD, chunk, 0)
    neg = jnp.full((16,), -1e30, jnp.float32)
    for g in range((PWD - EDV) // 16):
        zbuf[pl.ds(EDV + g * 16, 16)] = neg
    pltpu.sync_copy(zbuf, out_h.at[wid])


def _gnn_edge_sim(xl, xr, attp, mv, src_p, dst_p):
    src = src_p.reshape(-1)
    dst = dst_p.reshape(-1)
    e = jax.nn.leaky_relu(xl[src] + xr[dst], negative_slope=0.2) @ attp
    ex = jnp.exp(e - mv[0])
    val = ex[:, None] * xl[src]
    val = val.at[:, 20].set(ex)
    wid = jnp.arange(src.shape[0]) // PW
    cid = wid // NS
    seg = cid * NACC + dst
    return jax.ops.segment_sum(val, seg, num_segments=2 * NACC)


def _dec_sim(P, Q, c0, d2, rows_p, cols_p):
    rows = rows_p.reshape(NW, -1)
    cols = cols_p.reshape(NW, -1)
    h = jax.nn.relu(P[rows] + Q[cols] + c0)
    z = h @ d2
    mask = jnp.arange(PWD)[None, :] < EDV
    return jnp.where(mask, z, -1e30)


def _pad_w(w):
    din = w.shape[0]
    dinp = HP if din <= HP else din
    out = jnp.zeros((dinp, HP), jnp.float32)
    return out.at[:din, :w.shape[1]].set(w)


def _pad_v(v):
    return jnp.zeros((1, HP), jnp.float32).at[0, :v.shape[0]].set(v)


def kernel(x, edge_index, node_id, Wl1, bl1, Wr1, br1, att1, b1,
           Wl2, bl2, Wr2, br2, att2, b2, Wl3, bl3, Wr3, br3, att3, b3,
           D1w, D1b, D2w, D2b):
    i32 = edge_index.dtype
    loops = jnp.arange(N, dtype=i32)
    src = jnp.concatenate([edge_index[0], loops])
    dst = jnp.concatenate([edge_index[1], loops])
    pad = EPAD - EP
    src_p = jnp.concatenate([src, jnp.zeros((pad,), i32)]).reshape(NW, KCH, 128)
    dst_p = jnp.concatenate([dst, jnp.full((pad,), N + 8, i32)]).reshape(NW, KCH, 128)

    rows_p = jnp.pad(edge_index[0].reshape(NW, EDV),
                     ((0, 0), (0, PWD - EDV))).reshape(NW, KD, 128)
    cols_p = jnp.pad(edge_index[1].reshape(NW, EDV),
                     ((0, 0), (0, PWD - EDV))).reshape(NW, KD, 128)

    wl = [_pad_w(Wl1), _pad_w(Wl2), _pad_w(Wl3)]
    wr = [_pad_w(Wr1), _pad_w(Wr2), _pad_w(Wr3)]
    blv = [_pad_v(bl1), _pad_v(bl2), _pad_v(bl3)]
    brv = [_pad_v(br1), _pad_v(br2), _pad_v(br3)]
    bf = lambda a: a.astype(jnp.bfloat16).astype(jnp.float32)
    attp = [jnp.pad(bf(att1), (0, HP - H)), jnp.pad(bf(att2), (0, HP - H)),
            jnp.pad(bf(att3), (0, HP - H))]
    bv = [_pad_v(b1), _pad_v(b2), _pad_v(b3)]

    def mshift(cm, ap):
        return jnp.full((16,), jnp.sum(jnp.abs(ap) * (cm[0] + cm[1])),
                        jnp.float32)

    xl, xr, cm = _proj_first(x, wl[0], blv[0], wr[0], brv[0])
    acc1 = _gnn_edge(xl, xr, attp[0], mshift(cm, attp[0]), src_p, dst_p)
    x1p, xl, xr, cm = _combine_proj(acc1, bv[0], wl[1], blv[1], wr[1], brv[1])
    acc2 = _gnn_edge(xl, xr, attp[1], mshift(cm, attp[1]), src_p, dst_p)
    x2p, xl, xr, cm = _combine_proj(acc2, bv[1], wl[2], blv[2], wr[2], brv[2])
    acc3 = _gnn_edge(xl, xr, attp[2], mshift(cm, attp[2]), src_p, dst_p)

    dpad = jnp.zeros((HP, 64), jnp.float32)
    a1 = dpad.at[:H].set(D1w[0:20])
    a2 = dpad.at[:H].set(D1w[20:40])
    a3 = dpad.at[:H].set(D1w[40:60])
    c1 = dpad.at[:H].set(D1w[60:80])
    c2 = dpad.at[:H].set(D1w[80:100])
    c3 = dpad.at[:H].set(D1w[100:120])
    x3p, P, Q = _final_combine(acc3, bv[2], x1p, x2p, a1, a2, a3, c1, c2, c3)

    c0 = (jnp.dot(bf(x1p[node_id, :H]), bf(D1w[120:140]), precision=_HIGH)
          + jnp.dot(bf(x2p[node_id, :H]), bf(D1w[140:160]), precision=_HIGH)
          + jnp.dot(bf(x3p[node_id, :H]), bf(D1w[160:180]), precision=_HIGH)
          + D1b)
    z = _decoder_edges(P, Q, c0, bf(D2w[:, 0]), rows_p, cols_p)
    outp = _sparsemax(z)
    return outp[:, :EDV].reshape(E, 1)
